# Initial kernel scaffold; baseline (speedup 1.0000x reference)
import jax
import jax.numpy as jnp
from jax.experimental import pallas as pl


def kernel(z, W):
    raise NotImplementedError("write your pallas kernel here")


# speedup vs baseline: 1.0453x; 1.0453x over previous
import jax
import jax.numpy as jnp
from jax.experimental import pallas as pl
from jax.experimental.pallas import tpu as pltpu

_K = 1024
_D = 64
_BETA = 0.25
_BN = 2048
_N = 16384


def _vq_block(z_ref, w_ref, wt_ref, zq_ref, loss_ref):
    i = pl.program_id(0)
    zb = z_ref[...]
    w = w_ref[...]
    wt = wt_ref[...]
    z2 = jnp.sum(zb ** 2, axis=1, keepdims=True)
    w2 = jnp.sum(wt ** 2, axis=0, keepdims=True)
    s = jax.lax.dot_general(
        zb, w, (((1,), (1,)), ((), ())),
        preferred_element_type=jnp.float32)
    d2 = (z2 + w2) - 2.0 * s
    m = jnp.min(d2, axis=1, keepdims=True)
    iota = jax.lax.broadcasted_iota(jnp.int32, d2.shape, 1)
    idx = jnp.min(jnp.where(d2 == m, iota, _K),
                  axis=1, keepdims=True)
    onehot = (iota == idx).astype(jnp.float32)
    zq = jax.lax.dot_general(
        onehot, w, (((1,), (0,)), ((), ())),
        preferred_element_type=jnp.float32,
        precision=jax.lax.Precision.HIGHEST)
    zq_ref[...] = zb + (zq - zb)

    @pl.when(i == 0)
    def _init():
        loss_ref[...] = jnp.zeros_like(loss_ref)

    loss_ref[...] += jnp.sum(m) * ((1.0 + _BETA) / (_N * _D))


def kernel(z, W):
    zp = jnp.transpose(z, (0, 2, 3, 1))
    z_flat = zp.reshape(-1, _D)
    Wt = W.T
    zq_flat, loss = pl.pallas_call(
        _vq_block,
        grid=(_N // _BN,),
        in_specs=[
            pl.BlockSpec((_BN, _D), lambda i: (i, 0)),
            pl.BlockSpec((_K, _D), lambda i: (0, 0)),
            pl.BlockSpec((_D, _K), lambda i: (0, 0)),
        ],
        out_specs=[
            pl.BlockSpec((_BN, _D), lambda i: (i, 0)),
            pl.BlockSpec((1, 1), lambda i: (0, 0)),
        ],
        out_shape=[
            jax.ShapeDtypeStruct((_N, _D), jnp.float32),
            jax.ShapeDtypeStruct((1, 1), jnp.float32),
        ],
    )(z_flat, W, Wt)
    out = jnp.transpose(zq_flat.reshape(zp.shape), (0, 3, 1, 2))
    return out, loss[0, 0]
